# Initial kernel scaffold; baseline (speedup 1.0000x reference)
import jax
import jax.numpy as jnp
from jax.experimental import pallas as pl


def kernel(x, Wg_router, We_router, gate_w, up_w, down_w):
    raise NotImplementedError("write your pallas kernel here")


# speedup vs baseline: 1.9827x; 1.9827x over previous
import functools
import math

import jax
import jax.numpy as jnp
from jax import lax
from jax.experimental import pallas as pl
from jax.experimental.pallas import tpu as pltpu
from jax.experimental.pallas import tpu_sc as plsc

T = 2048
H = 1024
I = 2048
E = 16
G = 4
EPG = 4
TOPK = 2
CAP = int(math.ceil(T * TOPK / E * 1.25))
NSLOT = E * CAP
NW = 32
TPW = T // NW
HALF = TPW // 2
I_TILE = 512
N_ITILE = I // I_TILE

_NEG = -1e30


def _router_body(x_ref, w_ref, s0_ref, s1_ref, w0_ref, w1_ref):
    xf = x_ref[...]
    logits = jnp.dot(xf, w_ref[...], preferred_element_type=jnp.float32)
    gl = logits[:, :G]
    el = logits[:, G:]
    gmax = jnp.max(gl, axis=1, keepdims=True)
    gexp = jnp.exp(gl - gmax)
    gp = gexp / jnp.sum(gexp, axis=1, keepdims=True)
    gid = jnp.argmax(gp, axis=1)
    gprob = jnp.max(gp, axis=1)
    lane = lax.broadcasted_iota(jnp.int32, (T, E), 1)
    in_grp = (lane >> 2) == gid[:, None]
    elm = jnp.where(in_grp, el, _NEG)
    emax = jnp.max(elm, axis=1, keepdims=True)
    ee = jnp.where(in_grp, jnp.exp(elm - emax), 0.0)
    ep = ee / jnp.sum(ee, axis=1, keepdims=True)
    p1 = jnp.max(ep, axis=1)
    e0 = jnp.argmax(ep, axis=1)
    ep2 = jnp.where(lane == e0[:, None], -1.0, ep)
    p2 = jnp.max(ep2, axis=1)
    e1 = jnp.argmax(ep2, axis=1)
    psum = p1 + p2
    w0 = p1 / psum * gprob
    w1 = p2 / psum * gprob
    oh0 = (lane == e0[:, None]).astype(jnp.int32)
    oh1 = (lane == e1[:, None]).astype(jnp.int32)
    oh = oh0 + oh1
    c = oh
    sh = 1
    while sh < T:
        z = jnp.zeros((sh, E), jnp.int32)
        c = c + jnp.concatenate([z, c[:-sh]], axis=0)
        sh *= 2
    cex = c - oh
    pos0 = jnp.sum(cex * oh0, axis=1)
    pos1 = jnp.sum(cex * oh1, axis=1)
    keep0 = pos0 < CAP
    keep1 = pos1 < CAP
    s0_ref[...] = jnp.where(keep0, e0 * CAP + pos0, NSLOT)[:, None]
    s1_ref[...] = jnp.where(keep1, e1 * CAP + pos1, NSLOT)[:, None]
    ones = jnp.ones((1, 16), jnp.float32)
    w0_ref[...] = jnp.where(keep0, w0, 0.0)[:, None] * ones
    w1_ref[...] = jnp.where(keep1, w1, 0.0)[:, None] * ones


def _router(x, w_all):
    return pl.pallas_call(
        _router_body,
        out_shape=(
            jax.ShapeDtypeStruct((T, 1), jnp.int32),
            jax.ShapeDtypeStruct((T, 1), jnp.int32),
            jax.ShapeDtypeStruct((T, 16), jnp.float32),
            jax.ShapeDtypeStruct((T, 16), jnp.float32),
        ),
    )(x, w_all)


def _dispatch_body(x_hbm, s0_hbm, s1_hbm, buf_hbm, x_v, i0_v, i1_v, sem0, sem1):
    wid = lax.axis_index("s") * 2 + lax.axis_index("c")
    base = wid * TPW
    pltpu.sync_copy(x_hbm.at[pl.ds(base, TPW)], x_v)
    pltpu.sync_copy(s0_hbm.at[pl.ds(base, TPW)], i0_v)
    pltpu.sync_copy(s1_hbm.at[pl.ds(base, TPW)], i1_v)
    c0 = pltpu.async_copy(x_v, buf_hbm.at[i0_v], sem0)
    c1 = pltpu.async_copy(x_v, buf_hbm.at[i1_v], sem1)
    c0.wait()
    c1.wait()


def _dispatch(x, s0, s1):
    mesh = plsc.VectorSubcoreMesh(core_axis_name="c", subcore_axis_name="s")
    return pl.kernel(
        _dispatch_body,
        out_type=jax.ShapeDtypeStruct((NSLOT + 1, H), jnp.float32),
        mesh=mesh,
        scratch_types=[
            pltpu.VMEM((TPW, H), jnp.float32),
            pltpu.VMEM((TPW,), jnp.int32),
            pltpu.VMEM((TPW,), jnp.int32),
            pltpu.SemaphoreType.DMA,
            pltpu.SemaphoreType.DMA,
        ],
    )(x, s0, s1)


def _ffn_body(b_ref, gw_ref, uw_ref, dw_ref, o_ref):
    i = pl.program_id(1)
    xb = b_ref[...].astype(jnp.bfloat16)
    gw = gw_ref[0].astype(jnp.bfloat16)
    uw = uw_ref[0].astype(jnp.bfloat16)
    dw = dw_ref[0].astype(jnp.bfloat16)
    g = jnp.dot(xb, gw, preferred_element_type=jnp.float32)
    u = jnp.dot(xb, uw, preferred_element_type=jnp.float32)
    h = (g * jax.nn.sigmoid(g) * u).astype(jnp.bfloat16)
    contrib = jnp.dot(h, dw, preferred_element_type=jnp.float32)

    @pl.when(i == 0)
    def _():
        o_ref[...] = contrib

    @pl.when(i != 0)
    def _():
        o_ref[...] = o_ref[...] + contrib


def _ffn(buf, gate_w, up_w, down_w):
    return pl.pallas_call(
        _ffn_body,
        grid=(E, N_ITILE),
        in_specs=[
            pl.BlockSpec((CAP, H), lambda e, i: (e, 0)),
            pl.BlockSpec((1, H, I_TILE), lambda e, i: (e, 0, i)),
            pl.BlockSpec((1, H, I_TILE), lambda e, i: (e, 0, i)),
            pl.BlockSpec((1, I_TILE, H), lambda e, i: (e, i, 0)),
        ],
        out_specs=pl.BlockSpec((CAP, H), lambda e, i: (e, 0)),
        out_shape=jax.ShapeDtypeStruct((NSLOT + 1, H), jnp.float32),
        compiler_params=pltpu.CompilerParams(
            dimension_semantics=("arbitrary", "arbitrary"),
        ),
    )(buf, gate_w, up_w, down_w)


def _combine_body(eo_hbm, s0_hbm, s1_hbm, w0_hbm, w1_hbm, out_hbm,
                  idx_v, g_v, o_v, w_v, sem):
    wid = lax.axis_index("s") * 2 + lax.axis_index("c")
    for half in range(2):
        base = wid * TPW + half * HALF
        pltpu.sync_copy(s0_hbm.at[pl.ds(base, HALF)], idx_v)
        pltpu.async_copy(eo_hbm.at[idx_v], g_v, sem).wait()
        pltpu.sync_copy(w0_hbm.at[pl.ds(base, HALF)], w_v)

        def pass0(r, _):
            wv = w_v[r, :]
            m = wv != 0.0
            for cc in range(H // 16):
                seg = g_v[r, pl.ds(cc * 16, 16)]
                o_v[r, pl.ds(cc * 16, 16)] = jnp.where(m, seg * wv, 0.0)
            return 0

        lax.fori_loop(0, HALF, pass0, 0)
        pltpu.sync_copy(s1_hbm.at[pl.ds(base, HALF)], idx_v)
        pltpu.async_copy(eo_hbm.at[idx_v], g_v, sem).wait()
        pltpu.sync_copy(w1_hbm.at[pl.ds(base, HALF)], w_v)

        def pass1(r, _):
            wv = w_v[r, :]
            m = wv != 0.0
            for cc in range(H // 16):
                seg = g_v[r, pl.ds(cc * 16, 16)]
                acc = o_v[r, pl.ds(cc * 16, 16)]
                o_v[r, pl.ds(cc * 16, 16)] = acc + jnp.where(m, seg * wv, 0.0)
            return 0

        lax.fori_loop(0, HALF, pass1, 0)
        pltpu.sync_copy(o_v, out_hbm.at[pl.ds(base, HALF)])


def _combine(eo, s0, s1, w0, w1):
    mesh = plsc.VectorSubcoreMesh(core_axis_name="c", subcore_axis_name="s")
    return pl.kernel(
        _combine_body,
        out_type=jax.ShapeDtypeStruct((T, H), jnp.float32),
        mesh=mesh,
        scratch_types=[
            pltpu.VMEM((HALF,), jnp.int32),
            pltpu.VMEM((HALF, H), jnp.float32),
            pltpu.VMEM((HALF, H), jnp.float32),
            pltpu.VMEM((HALF, 16), jnp.float32),
            pltpu.SemaphoreType.DMA,
        ],
    )(eo, s0, s1, w0, w1)


def kernel(x, Wg_router, We_router, gate_w, up_w, down_w):
    w_all = jnp.concatenate(
        [Wg_router, jnp.transpose(We_router, (1, 0, 2)).reshape(H, E)], axis=1)
    s0, s1, w0, w1 = _router(x, w_all)
    s0 = s0.reshape(T)
    s1 = s1.reshape(T)
    buf = _dispatch(x, s0, s1)
    eo = _ffn(buf, gate_w, up_w, down_w)
    return _combine(eo, s0, s1, w0, w1)


# speedup vs baseline: 2.2690x; 1.1444x over previous
import functools
import math

import jax
import jax.numpy as jnp
from jax import lax
from jax.experimental import pallas as pl
from jax.experimental.pallas import tpu as pltpu
from jax.experimental.pallas import tpu_sc as plsc

T = 2048
H = 1024
I = 2048
E = 16
G = 4
EPG = 4
TOPK = 2
CAP = int(math.ceil(T * TOPK / E * 1.25))
NSLOT = E * CAP
NW = 32
TPW = T // NW
CH = 16
NCH = TPW // CH
I_TILE = 1024
N_ITILE = I // I_TILE

_NEG = -1e30


def _router_body(x_ref, w_ref, s0s_ref, s1s_ref, s0g_ref, s1g_ref,
                 w0_ref, w1_ref):
    xf = x_ref[...]
    logits = jnp.dot(xf, w_ref[...], preferred_element_type=jnp.float32)
    gl = logits[:, :G]
    el = logits[:, G:]
    gmax = jnp.max(gl, axis=1, keepdims=True)
    gexp = jnp.exp(gl - gmax)
    gp = gexp / jnp.sum(gexp, axis=1, keepdims=True)
    gid = jnp.argmax(gp, axis=1)
    gprob = jnp.max(gp, axis=1)
    lane = lax.broadcasted_iota(jnp.int32, (T, E), 1)
    in_grp = (lane >> 2) == gid[:, None]
    elm = jnp.where(in_grp, el, _NEG)
    emax = jnp.max(elm, axis=1, keepdims=True)
    ee = jnp.where(in_grp, jnp.exp(elm - emax), 0.0)
    ep = ee / jnp.sum(ee, axis=1, keepdims=True)
    p1 = jnp.max(ep, axis=1)
    e0 = jnp.argmax(ep, axis=1)
    ep2 = jnp.where(lane == e0[:, None], -1.0, ep)
    p2 = jnp.max(ep2, axis=1)
    e1 = jnp.argmax(ep2, axis=1)
    psum = p1 + p2
    w0 = p1 / psum * gprob
    w1 = p2 / psum * gprob
    oh0 = (lane == e0[:, None]).astype(jnp.int32)
    oh1 = (lane == e1[:, None]).astype(jnp.int32)
    oh = oh0 + oh1
    c = oh
    sh = 1
    while sh < T:
        z = jnp.zeros((sh, E), jnp.int32)
        c = c + jnp.concatenate([z, c[:-sh]], axis=0)
        sh *= 2
    cex = c - oh
    pos0 = jnp.sum(cex * oh0, axis=1)
    pos1 = jnp.sum(cex * oh1, axis=1)
    keep0 = pos0 < CAP
    keep1 = pos1 < CAP
    s0s_ref[...] = jnp.where(keep0, e0 * CAP + pos0, NSLOT)[:, None]
    s1s_ref[...] = jnp.where(keep1, e1 * CAP + pos1, NSLOT)[:, None]
    s0g_ref[...] = jnp.where(keep0, e0 * CAP + pos0, e0 * CAP)[:, None]
    s1g_ref[...] = jnp.where(keep1, e1 * CAP + pos1, e1 * CAP)[:, None]
    ones = jnp.ones((1, 16), jnp.float32)
    w0_ref[...] = jnp.where(keep0, w0, 0.0)[:, None] * ones
    w1_ref[...] = jnp.where(keep1, w1, 0.0)[:, None] * ones


def _router(x, w_all):
    return pl.pallas_call(
        _router_body,
        out_shape=(
            jax.ShapeDtypeStruct((T, 1), jnp.int32),
            jax.ShapeDtypeStruct((T, 1), jnp.int32),
            jax.ShapeDtypeStruct((T, 1), jnp.int32),
            jax.ShapeDtypeStruct((T, 1), jnp.int32),
            jax.ShapeDtypeStruct((T, 16), jnp.float32),
            jax.ShapeDtypeStruct((T, 16), jnp.float32),
        ),
    )(x, w_all)


def _dispatch_body(x_hbm, s0_hbm, s1_hbm, buf_hbm, x_v, i0_v, i1_v, sem0, sem1):
    wid = lax.axis_index("s") * 2 + lax.axis_index("c")
    base = wid * TPW
    pltpu.sync_copy(x_hbm.at[pl.ds(base, TPW)], x_v)
    pltpu.sync_copy(s0_hbm.at[pl.ds(base, TPW)], i0_v)
    pltpu.sync_copy(s1_hbm.at[pl.ds(base, TPW)], i1_v)
    c0 = pltpu.async_copy(x_v, buf_hbm.at[i0_v], sem0)
    c1 = pltpu.async_copy(x_v, buf_hbm.at[i1_v], sem1)
    c0.wait()
    c1.wait()


def _dispatch(x, s0, s1):
    mesh = plsc.VectorSubcoreMesh(core_axis_name="c", subcore_axis_name="s")
    return pl.kernel(
        _dispatch_body,
        out_type=jax.ShapeDtypeStruct((NSLOT + 1, H), jnp.float32),
        mesh=mesh,
        scratch_types=[
            pltpu.VMEM((TPW, H), jnp.float32),
            pltpu.VMEM((TPW,), jnp.int32),
            pltpu.VMEM((TPW,), jnp.int32),
            pltpu.SemaphoreType.DMA,
            pltpu.SemaphoreType.DMA,
        ],
    )(x, s0, s1)


def _ffn_body(b_ref, gw_ref, uw_ref, dw_ref, o_ref):
    i = pl.program_id(1)
    xb = b_ref[...].astype(jnp.bfloat16)
    gw = gw_ref[0].astype(jnp.bfloat16)
    uw = uw_ref[0].astype(jnp.bfloat16)
    dw = dw_ref[0].astype(jnp.bfloat16)
    g = jnp.dot(xb, gw, preferred_element_type=jnp.float32)
    u = jnp.dot(xb, uw, preferred_element_type=jnp.float32)
    h = (g * jax.nn.sigmoid(g) * u).astype(jnp.bfloat16)
    contrib = jnp.dot(h, dw, preferred_element_type=jnp.float32)

    @pl.when(i == 0)
    def _():
        o_ref[...] = contrib

    @pl.when(i != 0)
    def _():
        o_ref[...] = o_ref[...] + contrib


def _ffn(buf, gate_w, up_w, down_w):
    return pl.pallas_call(
        _ffn_body,
        grid=(E, N_ITILE),
        in_specs=[
            pl.BlockSpec((CAP, H), lambda e, i: (e, 0)),
            pl.BlockSpec((1, H, I_TILE), lambda e, i: (e, 0, i)),
            pl.BlockSpec((1, H, I_TILE), lambda e, i: (e, 0, i)),
            pl.BlockSpec((1, I_TILE, H), lambda e, i: (e, i, 0)),
        ],
        out_specs=pl.BlockSpec((CAP, H), lambda e, i: (e, 0)),
        out_shape=jax.ShapeDtypeStruct((NSLOT, H), jnp.float32),
        compiler_params=pltpu.CompilerParams(
            dimension_semantics=("arbitrary", "arbitrary"),
        ),
    )(buf, gate_w, up_w, down_w)


def _combine_body(eo_hbm, s0_hbm, s1_hbm, w0_hbm, w1_hbm, out_hbm,
                  w0_v, w1_v, ia0_v, ia1_v, ib0_v, ib1_v,
                  ga0_v, ga1_v, gb0_v, gb1_v, o0_v, o1_v, gsem, osem):
    wid = lax.axis_index("s") * 2 + lax.axis_index("c")
    base = wid * TPW
    pltpu.sync_copy(w0_hbm.at[pl.ds(base, TPW)], w0_v)
    pltpu.sync_copy(w1_hbm.at[pl.ds(base, TPW)], w1_v)
    ia = (ia0_v, ia1_v)
    ib = (ib0_v, ib1_v)
    ga = (ga0_v, ga1_v)
    gb = (gb0_v, gb1_v)
    ov = (o0_v, o1_v)

    def issue(c):
        p = c % 2
        pltpu.sync_copy(s0_hbm.at[pl.ds(base + c * CH, CH)], ia[p])
        pltpu.sync_copy(s1_hbm.at[pl.ds(base + c * CH, CH)], ib[p])
        da = pltpu.async_copy(eo_hbm.at[ia[p]], ga[p], gsem)
        db = pltpu.async_copy(eo_hbm.at[ib[p]], gb[p], gsem)
        return da, db

    pend = issue(0)
    owr = [None, None]
    for c in range(NCH):
        p = c % 2
        da, db = pend
        if c + 1 < NCH:
            pend = issue(c + 1)
        da.wait()
        db.wait()
        if owr[p] is not None:
            owr[p].wait()

        gac, gbc, oc = ga[p], gb[p], ov[p]

        def row(r, _):
            wa = w0_v[c * CH + r, :]
            wb = w1_v[c * CH + r, :]
            for s in range(H // 16):
                sl = pl.ds(s * 16, 16)
                oc[r, sl] = gac[r, sl] * wa + gbc[r, sl] * wb
            return 0

        lax.fori_loop(0, CH, row, 0)
        owr[p] = pltpu.async_copy(ov[p], out_hbm.at[pl.ds(base + c * CH, CH)],
                                  osem)
    for d in owr:
        if d is not None:
            d.wait()


def _combine(eo, s0, s1, w0, w1):
    mesh = plsc.VectorSubcoreMesh(core_axis_name="c", subcore_axis_name="s")
    return pl.kernel(
        _combine_body,
        out_type=jax.ShapeDtypeStruct((T, H), jnp.float32),
        mesh=mesh,
        scratch_types=[
            pltpu.VMEM((TPW, 16), jnp.float32),
            pltpu.VMEM((TPW, 16), jnp.float32),
            pltpu.VMEM((CH,), jnp.int32),
            pltpu.VMEM((CH,), jnp.int32),
            pltpu.VMEM((CH,), jnp.int32),
            pltpu.VMEM((CH,), jnp.int32),
            pltpu.VMEM((CH, H), jnp.float32),
            pltpu.VMEM((CH, H), jnp.float32),
            pltpu.VMEM((CH, H), jnp.float32),
            pltpu.VMEM((CH, H), jnp.float32),
            pltpu.VMEM((CH, H), jnp.float32),
            pltpu.VMEM((CH, H), jnp.float32),
            pltpu.SemaphoreType.DMA,
            pltpu.SemaphoreType.DMA,
        ],
    )(eo, s0, s1, w0, w1)


def kernel(x, Wg_router, We_router, gate_w, up_w, down_w):
    w_all = jnp.concatenate(
        [Wg_router, jnp.transpose(We_router, (1, 0, 2)).reshape(H, E)], axis=1)
    s0s, s1s, s0g, s1g, w0, w1 = _router(x, w_all)
    s0s = s0s.reshape(T)
    s1s = s1s.reshape(T)
    s0g = s0g.reshape(T)
    s1g = s1g.reshape(T)
    buf = _dispatch(x, s0s, s1s)
    eo = _ffn(buf, gate_w, up_w, down_w)
    return _combine(eo, s0g, s1g, w0, w1)


# speedup vs baseline: 2.2725x; 1.0015x over previous
import functools
import math

import jax
import jax.numpy as jnp
from jax import lax
from jax.experimental import pallas as pl
from jax.experimental.pallas import tpu as pltpu
from jax.experimental.pallas import tpu_sc as plsc

T = 2048
H = 1024
I = 2048
E = 16
G = 4
EPG = 4
TOPK = 2
CAP = int(math.ceil(T * TOPK / E * 1.25))
NSLOT = E * CAP
NW = 32
TPW = T // NW
CH = 16
NCH = TPW // CH
I_TILE = 2048
N_ITILE = I // I_TILE

_NEG = -1e30


def _router_body(x_ref, w_ref, s0s_ref, s1s_ref, s0g_ref, s1g_ref,
                 w0_ref, w1_ref):
    xf = x_ref[...]
    logits = jnp.dot(xf, w_ref[...], preferred_element_type=jnp.float32)
    gl = logits[:, :G]
    el = logits[:, G:]
    gmax = jnp.max(gl, axis=1, keepdims=True)
    gexp = jnp.exp(gl - gmax)
    gp = gexp / jnp.sum(gexp, axis=1, keepdims=True)
    gid = jnp.argmax(gp, axis=1)
    gprob = jnp.max(gp, axis=1)
    lane = lax.broadcasted_iota(jnp.int32, (T, E), 1)
    in_grp = (lane >> 2) == gid[:, None]
    elm = jnp.where(in_grp, el, _NEG)
    emax = jnp.max(elm, axis=1, keepdims=True)
    ee = jnp.where(in_grp, jnp.exp(elm - emax), 0.0)
    ep = ee / jnp.sum(ee, axis=1, keepdims=True)
    p1 = jnp.max(ep, axis=1)
    e0 = jnp.argmax(ep, axis=1)
    ep2 = jnp.where(lane == e0[:, None], -1.0, ep)
    p2 = jnp.max(ep2, axis=1)
    e1 = jnp.argmax(ep2, axis=1)
    psum = p1 + p2
    w0 = p1 / psum * gprob
    w1 = p2 / psum * gprob
    oh0 = (lane == e0[:, None]).astype(jnp.int32)
    oh1 = (lane == e1[:, None]).astype(jnp.int32)
    oh = oh0 + oh1
    c = oh
    sh = 1
    while sh < T:
        z = jnp.zeros((sh, E), jnp.int32)
        c = c + jnp.concatenate([z, c[:-sh]], axis=0)
        sh *= 2
    cex = c - oh
    pos0 = jnp.sum(cex * oh0, axis=1)
    pos1 = jnp.sum(cex * oh1, axis=1)
    keep0 = pos0 < CAP
    keep1 = pos1 < CAP
    s0s_ref[...] = jnp.where(keep0, e0 * CAP + pos0, NSLOT)[:, None]
    s1s_ref[...] = jnp.where(keep1, e1 * CAP + pos1, NSLOT)[:, None]
    s0g_ref[...] = jnp.where(keep0, e0 * CAP + pos0, e0 * CAP)[:, None]
    s1g_ref[...] = jnp.where(keep1, e1 * CAP + pos1, e1 * CAP)[:, None]
    ones = jnp.ones((1, 16), jnp.float32)
    w0_ref[...] = jnp.where(keep0, w0, 0.0)[:, None] * ones
    w1_ref[...] = jnp.where(keep1, w1, 0.0)[:, None] * ones


def _router(x, w_all):
    return pl.pallas_call(
        _router_body,
        out_shape=(
            jax.ShapeDtypeStruct((T, 1), jnp.int32),
            jax.ShapeDtypeStruct((T, 1), jnp.int32),
            jax.ShapeDtypeStruct((T, 1), jnp.int32),
            jax.ShapeDtypeStruct((T, 1), jnp.int32),
            jax.ShapeDtypeStruct((T, 16), jnp.float32),
            jax.ShapeDtypeStruct((T, 16), jnp.float32),
        ),
    )(x, w_all)


def _dispatch_body(x_hbm, s0_hbm, s1_hbm, buf_hbm, x_v, i0_v, i1_v, sem0, sem1):
    wid = lax.axis_index("s") * 2 + lax.axis_index("c")
    base = wid * TPW
    pltpu.sync_copy(x_hbm.at[pl.ds(base, TPW)], x_v)
    pltpu.sync_copy(s0_hbm.at[pl.ds(base, TPW)], i0_v)
    pltpu.sync_copy(s1_hbm.at[pl.ds(base, TPW)], i1_v)
    c0 = pltpu.async_copy(x_v, buf_hbm.at[i0_v], sem0)
    c1 = pltpu.async_copy(x_v, buf_hbm.at[i1_v], sem1)
    c0.wait()
    c1.wait()


def _dispatch(x, s0, s1):
    mesh = plsc.VectorSubcoreMesh(core_axis_name="c", subcore_axis_name="s")
    return pl.kernel(
        _dispatch_body,
        out_type=jax.ShapeDtypeStruct((NSLOT + 1, H), jnp.float32),
        mesh=mesh,
        scratch_types=[
            pltpu.VMEM((TPW, H), jnp.float32),
            pltpu.VMEM((TPW,), jnp.int32),
            pltpu.VMEM((TPW,), jnp.int32),
            pltpu.SemaphoreType.DMA,
            pltpu.SemaphoreType.DMA,
        ],
    )(x, s0, s1)


def _ffn_body(b_ref, gw_ref, uw_ref, dw_ref, o_ref):
    i = pl.program_id(1)
    xb = b_ref[...].astype(jnp.bfloat16)
    gw = gw_ref[0].astype(jnp.bfloat16)
    uw = uw_ref[0].astype(jnp.bfloat16)
    dw = dw_ref[0].astype(jnp.bfloat16)
    g = jnp.dot(xb, gw, preferred_element_type=jnp.float32)
    u = jnp.dot(xb, uw, preferred_element_type=jnp.float32)
    h = (g * jax.nn.sigmoid(g) * u).astype(jnp.bfloat16)
    contrib = jnp.dot(h, dw, preferred_element_type=jnp.float32)

    @pl.when(i == 0)
    def _():
        o_ref[...] = contrib

    @pl.when(i != 0)
    def _():
        o_ref[...] = o_ref[...] + contrib


def _ffn(buf, gate_w, up_w, down_w):
    return pl.pallas_call(
        _ffn_body,
        grid=(E, N_ITILE),
        in_specs=[
            pl.BlockSpec((CAP, H), lambda e, i: (e, 0)),
            pl.BlockSpec((1, H, I_TILE), lambda e, i: (e, 0, i)),
            pl.BlockSpec((1, H, I_TILE), lambda e, i: (e, 0, i)),
            pl.BlockSpec((1, I_TILE, H), lambda e, i: (e, i, 0)),
        ],
        out_specs=pl.BlockSpec((CAP, H), lambda e, i: (e, 0)),
        out_shape=jax.ShapeDtypeStruct((NSLOT, H), jnp.float32),
        compiler_params=pltpu.CompilerParams(
            dimension_semantics=("arbitrary", "arbitrary"),
        ),
    )(buf, gate_w, up_w, down_w)


def _combine_body(eo_hbm, s0_hbm, s1_hbm, w0_hbm, w1_hbm, out_hbm,
                  w0_v, w1_v, ia0_v, ia1_v, ib0_v, ib1_v,
                  ga0_v, ga1_v, gb0_v, gb1_v, o0_v, o1_v, gsem, osem):
    wid = lax.axis_index("s") * 2 + lax.axis_index("c")
    base = wid * TPW
    pltpu.sync_copy(w0_hbm.at[pl.ds(base, TPW)], w0_v)
    pltpu.sync_copy(w1_hbm.at[pl.ds(base, TPW)], w1_v)
    ia = (ia0_v, ia1_v)
    ib = (ib0_v, ib1_v)
    ga = (ga0_v, ga1_v)
    gb = (gb0_v, gb1_v)
    ov = (o0_v, o1_v)

    def issue(c):
        p = c % 2
        pltpu.sync_copy(s0_hbm.at[pl.ds(base + c * CH, CH)], ia[p])
        pltpu.sync_copy(s1_hbm.at[pl.ds(base + c * CH, CH)], ib[p])
        da = pltpu.async_copy(eo_hbm.at[ia[p]], ga[p], gsem)
        db = pltpu.async_copy(eo_hbm.at[ib[p]], gb[p], gsem)
        return da, db

    pend = issue(0)
    owr = [None, None]
    for c in range(NCH):
        p = c % 2
        da, db = pend
        if c + 1 < NCH:
            pend = issue(c + 1)
        da.wait()
        db.wait()
        if owr[p] is not None:
            owr[p].wait()

        gac, gbc, oc = ga[p], gb[p], ov[p]

        def row(r, _):
            wa = w0_v[c * CH + r, :]
            wb = w1_v[c * CH + r, :]
            for s in range(H // 16):
                sl = pl.ds(s * 16, 16)
                oc[r, sl] = gac[r, sl] * wa + gbc[r, sl] * wb
            return 0

        lax.fori_loop(0, CH, row, 0)
        owr[p] = pltpu.async_copy(ov[p], out_hbm.at[pl.ds(base + c * CH, CH)],
                                  osem)
    for d in owr:
        if d is not None:
            d.wait()


def _combine(eo, s0, s1, w0, w1):
    mesh = plsc.VectorSubcoreMesh(core_axis_name="c", subcore_axis_name="s")
    return pl.kernel(
        _combine_body,
        out_type=jax.ShapeDtypeStruct((T, H), jnp.float32),
        mesh=mesh,
        scratch_types=[
            pltpu.VMEM((TPW, 16), jnp.float32),
            pltpu.VMEM((TPW, 16), jnp.float32),
            pltpu.VMEM((CH,), jnp.int32),
            pltpu.VMEM((CH,), jnp.int32),
            pltpu.VMEM((CH,), jnp.int32),
            pltpu.VMEM((CH,), jnp.int32),
            pltpu.VMEM((CH, H), jnp.float32),
            pltpu.VMEM((CH, H), jnp.float32),
            pltpu.VMEM((CH, H), jnp.float32),
            pltpu.VMEM((CH, H), jnp.float32),
            pltpu.VMEM((CH, H), jnp.float32),
            pltpu.VMEM((CH, H), jnp.float32),
            pltpu.SemaphoreType.DMA,
            pltpu.SemaphoreType.DMA,
        ],
    )(eo, s0, s1, w0, w1)


def kernel(x, Wg_router, We_router, gate_w, up_w, down_w):
    w_all = jnp.concatenate(
        [Wg_router, jnp.transpose(We_router, (1, 0, 2)).reshape(H, E)], axis=1)
    s0s, s1s, s0g, s1g, w0, w1 = _router(x, w_all)
    s0s = s0s.reshape(T)
    s1s = s1s.reshape(T)
    s0g = s0g.reshape(T)
    s1g = s1g.reshape(T)
    buf = _dispatch(x, s0s, s1s)
    eo = _ffn(buf, gate_w, up_w, down_w)
    return _combine(eo, s0g, s1g, w0, w1)
